# baseline (speedup 1.0000x reference)
import functools

import jax
import jax.numpy as jnp
from jax import lax
from jax.experimental import pallas as pl
from jax.experimental.pallas import tpu as pltpu
from jax.experimental.pallas import tpu_sc as plsc

B = 32
P = 20000
C = 5
NC = 2
NS = 16
L = 16
CH = 2000
NCHUNK = P // CH
GRP = CH // L

_LN2 = 0.6931471805599453


def _sc_log(s):
    bits = lax.bitcast_convert_type(s, jnp.int32)
    e = lax.shift_right_logical(bits, 23) - 127
    mant = lax.bitcast_convert_type(
        jnp.bitwise_or(jnp.bitwise_and(bits, 0x007FFFFF), 0x3F800000),
        jnp.float32,
    )
    z = (mant - 1.0) / (mant + 1.0)
    z2 = z * z
    poly = 1.0 + z2 * (
        (1.0 / 3.0) + z2 * ((1.0 / 5.0) + z2 * ((1.0 / 7.0) + z2 * (1.0 / 9.0)))
    )
    return e.astype(jnp.float32) * _LN2 + 2.0 * z * poly


def _sc_body(conf_hbm, lab_hbm, loc_hbm, tloc_hbm, part_hbm, mloss_hbm,
             cbuf, lbuf, pbuf, tbuf, mbuf, obuf):
    wid = lax.axis_index("s") * NC + lax.axis_index("c")
    iota = lax.iota(jnp.int32, L)
    idx5 = iota * C
    idx4 = iota * 4
    zero = jnp.zeros((L,), jnp.float32)
    npos_a, wsum_a, negsum_a, locsum_a = zero, zero, zero, zero

    for ch in range(NCHUNK):
        co = wid * (P * C) + ch * (CH * C)
        po = wid * (P * 4) + ch * (CH * 4)
        lo = wid * P + ch * CH
        pltpu.sync_copy(conf_hbm.at[pl.ds(co, CH * C)], cbuf)
        pltpu.sync_copy(lab_hbm.at[pl.ds(lo, CH)], lbuf)
        pltpu.sync_copy(loc_hbm.at[pl.ds(po, CH * 4)], pbuf)
        pltpu.sync_copy(tloc_hbm.at[pl.ds(po, CH * 4)], tbuf)

        def group(g, carry):
            npos, wsum, negsum, locsum = carry
            base_c = g * (L * C)
            x = [plsc.load_gather(cbuf, [idx5 + (base_c + c)]) for c in range(C)]
            lab = lbuf[pl.ds(g * L, L)]
            m = jnp.maximum(jnp.maximum(jnp.maximum(x[0], x[1]),
                                        jnp.maximum(x[2], x[3])), x[4])
            s = (jnp.exp(x[0] - m) + jnp.exp(x[1] - m) + jnp.exp(x[2] - m)
                 + jnp.exp(x[3] - m) + jnp.exp(x[4] - m))
            lse = m + _sc_log(s)
            xl = jnp.where(lab == 1, x[1], x[0])
            xl = jnp.where(lab == 2, x[2], xl)
            xl = jnp.where(lab == 3, x[3], xl)
            xl = jnp.where(lab == 4, x[4], xl)
            w = jnp.where(lab == 1, 1.0, 0.05)
            w = jnp.where(lab == 2, 5.0, w)
            w = jnp.where(lab == 3, 20.0, w)
            w = jnp.where(lab == 4, 10.0, w)
            posf = jnp.where(lab > 0, 1.0, 0.0)
            nll = lse - xl
            l0 = lse - x[0]
            npos = npos + posf
            wsum = wsum + posf * (w * nll)
            negsum = negsum + (1.0 - posf) * l0
            mbuf[pl.ds(g * L, L)] = jnp.where(lab > 0, -1.0, l0)
            return npos, wsum, negsum, locsum

        npos_a, wsum_a, negsum_a, locsum_a = lax.fori_loop(
            0, GRP, group, (npos_a, wsum_a, negsum_a, locsum_a))

        cmask = jnp.where(jnp.bitwise_and(iota, 3) >= 2, 1.0, 0.0)

        def locgroup(g, locsum):
            base = g * L
            lv = pbuf[pl.ds(base, L)]
            tv = tbuf[pl.ds(base, L)]
            plab = plsc.load_gather(lbuf, [lax.shift_right_logical(iota, 2) + g * 4])
            pm = jnp.where(plab > 0, 1.0, 0.0)
            d = lv - tv
            e = jnp.abs(jnp.exp(lv) - jnp.exp(tv))
            return locsum + pm * (d * d + cmask * e)

        locsum_a = lax.fori_loop(0, (CH * 4) // L, locgroup, locsum_a)
        pltpu.sync_copy(mbuf, mloss_hbm.at[pl.ds(lo, CH)])

    obuf[pl.ds(0, L)] = npos_a
    obuf[pl.ds(L, L)] = wsum_a
    obuf[pl.ds(2 * L, L)] = negsum_a
    obuf[pl.ds(3 * L, L)] = locsum_a
    pltpu.sync_copy(obuf, part_hbm.at[pl.ds(wid * (4 * L), 4 * L)])


@functools.cache
def _sc_main():
    return pl.kernel(
        _sc_body,
        out_type=(
            jax.ShapeDtypeStruct((B * 4 * L,), jnp.float32),
            jax.ShapeDtypeStruct((B * P,), jnp.float32),
        ),
        mesh=plsc.VectorSubcoreMesh(core_axis_name="c", subcore_axis_name="s",
                                    num_cores=NC, num_subcores=NS),
        scratch_types=[
            pltpu.VMEM((CH * C,), jnp.float32),
            pltpu.VMEM((CH,), jnp.int32),
            pltpu.VMEM((CH * 4,), jnp.float32),
            pltpu.VMEM((CH * 4,), jnp.float32),
            pltpu.VMEM((CH,), jnp.float32),
            pltpu.VMEM((4 * L,), jnp.float32),
        ],
        compiler_params=pltpu.CompilerParams(needs_layout_passes=False),
    )


def _topk_body(mloss_ref, k_ref, nneg_ref, out_ref):
    v = mloss_ref[...]
    neg = v >= 0.0
    k_eff = jnp.minimum(k_ref[0, 0], nneg_ref[...])

    def bit_step(i, t_bits):
        cand = jnp.bitwise_or(t_bits, lax.shift_left(jnp.int32(1), 30 - i))
        t = lax.bitcast_convert_type(cand, jnp.float32)
        cnt = jnp.sum(jnp.where(neg & (v >= t), 1.0, 0.0), axis=1, keepdims=True)
        return jnp.where(cnt >= k_eff, cand, t_bits)

    t_bits = lax.fori_loop(0, 31, bit_step, jnp.zeros((B, 1), jnp.int32))
    t = lax.bitcast_convert_type(t_bits, jnp.float32)
    sel = neg & (v > t)
    cgt = jnp.sum(jnp.where(sel, 1.0, 0.0), axis=1, keepdims=True)
    ssum = jnp.sum(jnp.where(sel, v, 0.0), axis=1, keepdims=True)
    rows = ssum + (k_eff - cgt) * t
    rows = jnp.where(k_eff > 0.0, rows, 0.0)
    out_ref[...] = jnp.sum(rows).reshape(1, 1)


def _topk_sum(mloss2d, kf, nneg):
    return pl.pallas_call(
        _topk_body,
        out_shape=jax.ShapeDtypeStruct((1, 1), jnp.float32),
    )(mloss2d, kf.reshape(1, 1), nneg.reshape(B, 1))[0, 0]


def kernel(confidence, locations, target_confidence, target_locations):
    conf = confidence.reshape(-1)
    lab = target_confidence.reshape(-1)
    loc = locations.reshape(-1)
    tloc = target_locations.reshape(-1)
    partials, mloss = _sc_main()(conf, lab, loc, tloc)
    part = partials.reshape(B, 4, L)
    npos_rows = jnp.sum(part[:, 0, :], axis=1)
    num_pos = jnp.sum(npos_rows)
    wsum = jnp.sum(part[:, 1, :])
    negsum_rows = jnp.sum(part[:, 2, :], axis=1)
    locsum = jnp.sum(part[:, 3, :])
    kf = 3.0 * num_pos
    neg_contrib = lax.cond(
        kf >= float(P),
        lambda: jnp.sum(negsum_rows),
        lambda: _topk_sum(mloss.reshape(B, P), kf, float(P) - npos_rows),
    )
    return (locsum + wsum + 0.05 * neg_contrib) / num_pos


# speedup vs baseline: 30.4582x; 30.4582x over previous
import functools

import jax
import jax.numpy as jnp
from jax import lax
from jax.experimental import pallas as pl
from jax.experimental.pallas import tpu as pltpu
from jax.experimental.pallas import tpu_sc as plsc

B = 32
P = 20000
C = 5
ROWP = 20096
NC = 2
NS = 16
L = 16
CHS = 4000
NCH = P // CHS
GRP = CHS // L
RPB = 8


def _prep_body(ct_ref, lt_ref, tlt_ref, lab_ref, m_ref, wpart_ref, lpart_ref):
    lab = lab_ref[...]
    pos = lab > 0
    posf = jnp.where(pos, 1.0, 0.0)
    x = [ct_ref[c] for c in range(C)]
    mx = jnp.maximum(jnp.maximum(jnp.maximum(x[0], x[1]),
                                 jnp.maximum(x[2], x[3])), x[4])
    s = (jnp.exp(x[0] - mx) + jnp.exp(x[1] - mx) + jnp.exp(x[2] - mx)
         + jnp.exp(x[3] - mx) + jnp.exp(x[4] - mx))
    lse = mx + jnp.log(s)
    xl = jnp.where(lab == 1, x[1], x[0])
    xl = jnp.where(lab == 2, x[2], xl)
    xl = jnp.where(lab == 3, x[3], xl)
    xl = jnp.where(lab == 4, x[4], xl)
    w = jnp.where(lab == 1, 1.0, 0.05)
    w = jnp.where(lab == 2, 5.0, w)
    w = jnp.where(lab == 3, 20.0, w)
    w = jnp.where(lab == 4, 10.0, w)
    wtot = jnp.sum(posf * (w * (lse - xl)))
    mvals = jnp.where(pos, -1.0, lse - x[0])
    for r in range(RPB):
        m_ref[pl.ds(r * ROWP, P)] = mvals[r, :]
        m_ref[pl.ds(r * ROWP + P, ROWP - P)] = jnp.full((ROWP - P,), -1.0,
                                                        jnp.float32)
    d = lt_ref[...] - tlt_ref[...]
    sq = jnp.sum(d * d, axis=1)
    e = jnp.abs(jnp.exp(lt_ref[:, 2:4, :]) - jnp.exp(tlt_ref[:, 2:4, :]))
    ltot = jnp.sum((sq + jnp.sum(e, axis=1)) * posf)
    lane0 = jax.lax.broadcasted_iota(jnp.int32, (1, 1, 128), 2) == 0
    wpart_ref[...] = jnp.where(lane0, wtot, 0.0)
    lpart_ref[...] = jnp.where(lane0, ltot, 0.0)


def _prep(ct, lt, tlt, lab):
    nsteps = B // RPB
    return pl.pallas_call(
        _prep_body,
        grid=(nsteps,),
        in_specs=[
            pl.BlockSpec((C, RPB, P), lambda i: (0, i, 0)),
            pl.BlockSpec((RPB, 4, P), lambda i: (i, 0, 0)),
            pl.BlockSpec((RPB, 4, P), lambda i: (i, 0, 0)),
            pl.BlockSpec((RPB, P), lambda i: (i, 0)),
        ],
        out_specs=[
            pl.BlockSpec((RPB * ROWP,), lambda i: (i,)),
            pl.BlockSpec((1, 1, 128), lambda i: (i, 0, 0)),
            pl.BlockSpec((1, 1, 128), lambda i: (i, 0, 0)),
        ],
        out_shape=[
            jax.ShapeDtypeStruct((B * ROWP,), jnp.float32),
            jax.ShapeDtypeStruct((nsteps, 1, 128), jnp.float32),
            jax.ShapeDtypeStruct((nsteps, 1, 128), jnp.float32),
        ],
    )(ct, lt, tlt, lab)


def _sc_body(m_hbm, part_hbm, mbuf, obuf):
    wid = lax.axis_index("s") * NC + lax.axis_index("c")
    zero = jnp.zeros((L,), jnp.float32)
    npos_a, negsum_a = zero, zero

    for ch in range(NCH):
        pltpu.sync_copy(m_hbm.at[pl.ds(wid * ROWP + ch * CHS, CHS)], mbuf)

        def group(g, carry):
            npos, negsum = carry
            v = mbuf[pl.ds(g * L, L)]
            isneg = v >= 0.0
            npos = npos + jnp.where(isneg, 0.0, 1.0)
            negsum = negsum + jnp.where(isneg, v, 0.0)
            return npos, negsum

        npos_a, negsum_a = lax.fori_loop(0, GRP, group, (npos_a, negsum_a))

    obuf[pl.ds(0, L)] = npos_a
    obuf[pl.ds(L, L)] = negsum_a
    pltpu.sync_copy(obuf, part_hbm.at[pl.ds(wid * (2 * L), 2 * L)])


@functools.cache
def _sc_main():
    return pl.kernel(
        _sc_body,
        out_type=jax.ShapeDtypeStruct((B * 2 * L,), jnp.float32),
        mesh=plsc.VectorSubcoreMesh(core_axis_name="c", subcore_axis_name="s",
                                    num_cores=NC, num_subcores=NS),
        scratch_types=[
            pltpu.VMEM((CHS,), jnp.float32),
            pltpu.VMEM((2 * L,), jnp.float32),
        ],
        compiler_params=pltpu.CompilerParams(needs_layout_passes=False),
    )


def _topk_body(m_ref, k_ref, nneg_ref, out_ref):
    v = m_ref[...]
    neg = v >= 0.0
    k_eff = jnp.minimum(k_ref[0, 0], nneg_ref[...])

    def bit_step(i, t_bits):
        cand = jnp.bitwise_or(t_bits, lax.shift_left(jnp.int32(1), 30 - i))
        t = lax.bitcast_convert_type(cand, jnp.float32)
        cnt = jnp.sum(jnp.where(neg & (v >= t), 1.0, 0.0), axis=1, keepdims=True)
        return jnp.where(cnt >= k_eff, cand, t_bits)

    t_bits = lax.fori_loop(0, 31, bit_step, jnp.zeros((B, 1), jnp.int32))
    t = lax.bitcast_convert_type(t_bits, jnp.float32)
    sel = neg & (v > t)
    cgt = jnp.sum(jnp.where(sel, 1.0, 0.0), axis=1, keepdims=True)
    ssum = jnp.sum(jnp.where(sel, v, 0.0), axis=1, keepdims=True)
    rows = ssum + (k_eff - cgt) * t
    rows = jnp.where(k_eff > 0.0, rows, 0.0)
    out_ref[...] = jnp.sum(rows).reshape(1, 1)


def _topk_sum(m2d, kf, nneg):
    return pl.pallas_call(
        _topk_body,
        out_shape=jax.ShapeDtypeStruct((1, 1), jnp.float32),
    )(m2d, kf.reshape(1, 1), nneg.reshape(B, 1))[0, 0]


def kernel(confidence, locations, target_confidence, target_locations):
    ct = jnp.transpose(confidence, (2, 0, 1))
    lt = jnp.transpose(locations, (0, 2, 1))
    tlt = jnp.transpose(target_locations, (0, 2, 1))
    m1d, wpart, lpart = _prep(ct, lt, tlt, target_confidence)
    partials = _sc_main()(m1d)
    part = partials.reshape(B, 2, L)
    npos_rows = jnp.sum(part[:, 0, :], axis=1)
    num_pos = jnp.sum(npos_rows)
    negsum_rows = jnp.sum(part[:, 1, :], axis=1)
    wsum = jnp.sum(wpart)
    locsum = jnp.sum(lpart)
    kf = 3.0 * num_pos
    neg_contrib = lax.cond(
        kf >= float(P),
        lambda: jnp.sum(negsum_rows),
        lambda: _topk_sum(m1d.reshape(B, ROWP), kf, float(P) - npos_rows),
    )
    return (locsum + wsum + 0.05 * neg_contrib) / num_pos


# speedup vs baseline: 32.9745x; 1.0826x over previous
import functools

import jax
import jax.numpy as jnp
from jax import lax
from jax.experimental import pallas as pl
from jax.experimental.pallas import tpu as pltpu
from jax.experimental.pallas import tpu_sc as plsc

B = 32
P = 20000
C = 5
ROWP = 20096
NC = 2
NS = 16
L = 16
CHS = 4000
NCH = P // CHS
GRP = CHS // L
RPB = 8


def _prep_body(ct_ref, lt_ref, tlt_ref, lab_ref, m_ref, wpart_ref, lpart_ref):
    lab = lab_ref[...]
    pos = lab > 0
    posf = jnp.where(pos, 1.0, 0.0)
    x = [ct_ref[c] for c in range(C)]
    mx = jnp.maximum(jnp.maximum(jnp.maximum(x[0], x[1]),
                                 jnp.maximum(x[2], x[3])), x[4])
    s = (jnp.exp(x[0] - mx) + jnp.exp(x[1] - mx) + jnp.exp(x[2] - mx)
         + jnp.exp(x[3] - mx) + jnp.exp(x[4] - mx))
    lse = mx + jnp.log(s)
    xl = jnp.where(lab == 1, x[1], x[0])
    xl = jnp.where(lab == 2, x[2], xl)
    xl = jnp.where(lab == 3, x[3], xl)
    xl = jnp.where(lab == 4, x[4], xl)
    w = jnp.where(lab == 1, 1.0, 0.05)
    w = jnp.where(lab == 2, 5.0, w)
    w = jnp.where(lab == 3, 20.0, w)
    w = jnp.where(lab == 4, 10.0, w)
    wtot = jnp.sum(posf * (w * (lse - xl)))
    mvals = jnp.where(pos, -1.0, lse - x[0])
    mp = jnp.concatenate(
        [mvals, jnp.full((RPB, ROWP - P), -1.0, jnp.float32)], axis=1)
    for r in range(RPB):
        m_ref[pl.ds(r * ROWP, ROWP)] = mp[r, :]
    d = lt_ref[...] - tlt_ref[...]
    sq = jnp.sum(d * d, axis=1)
    e = jnp.abs(jnp.exp(lt_ref[:, 2:4, :]) - jnp.exp(tlt_ref[:, 2:4, :]))
    ltot = jnp.sum((sq + jnp.sum(e, axis=1)) * posf)
    lane0 = jax.lax.broadcasted_iota(jnp.int32, (1, 1, 128), 2) == 0
    wpart_ref[...] = jnp.where(lane0, wtot, 0.0)
    lpart_ref[...] = jnp.where(lane0, ltot, 0.0)


def _prep(ct, lt, tlt, lab):
    nsteps = B // RPB
    return pl.pallas_call(
        _prep_body,
        grid=(nsteps,),
        in_specs=[
            pl.BlockSpec((C, RPB, P), lambda i: (0, i, 0)),
            pl.BlockSpec((RPB, 4, P), lambda i: (i, 0, 0)),
            pl.BlockSpec((RPB, 4, P), lambda i: (i, 0, 0)),
            pl.BlockSpec((RPB, P), lambda i: (i, 0)),
        ],
        out_specs=[
            pl.BlockSpec((RPB * ROWP,), lambda i: (i,)),
            pl.BlockSpec((1, 1, 128), lambda i: (i, 0, 0)),
            pl.BlockSpec((1, 1, 128), lambda i: (i, 0, 0)),
        ],
        out_shape=[
            jax.ShapeDtypeStruct((B * ROWP,), jnp.float32),
            jax.ShapeDtypeStruct((nsteps, 1, 128), jnp.float32),
            jax.ShapeDtypeStruct((nsteps, 1, 128), jnp.float32),
        ],
    )(ct, lt, tlt, lab)


_UNROLL = 4
_NITER = ROWP // (L * _UNROLL)


def _sc_body(m_hbm, part_hbm, mbuf, obuf):
    wid = lax.axis_index("s") * NC + lax.axis_index("c")
    zero = jnp.zeros((L,), jnp.float32)
    pltpu.sync_copy(m_hbm.at[pl.ds(wid * ROWP, ROWP)], mbuf)

    def group(g, carry):
        base = g * (L * _UNROLL)
        out = []
        for u in range(_UNROLL):
            npos_u, negsum_u = carry[2 * u], carry[2 * u + 1]
            v = mbuf[pl.ds(base + u * L, L)]
            isneg = v >= 0.0
            out.append(npos_u + jnp.where(isneg, 0.0, 1.0))
            out.append(negsum_u + jnp.where(isneg, v, 0.0))
        return tuple(out)

    acc = lax.fori_loop(0, _NITER, group, (zero,) * (2 * _UNROLL))
    npos_a = acc[0] + acc[2] + acc[4] + acc[6]
    negsum_a = acc[1] + acc[3] + acc[5] + acc[7]
    obuf[pl.ds(0, L)] = npos_a
    obuf[pl.ds(L, L)] = negsum_a
    pltpu.sync_copy(obuf, part_hbm.at[pl.ds(wid * (2 * L), 2 * L)])


@functools.cache
def _sc_main():
    return pl.kernel(
        _sc_body,
        out_type=jax.ShapeDtypeStruct((B * 2 * L,), jnp.float32),
        mesh=plsc.VectorSubcoreMesh(core_axis_name="c", subcore_axis_name="s",
                                    num_cores=NC, num_subcores=NS),
        scratch_types=[
            pltpu.VMEM((ROWP,), jnp.float32),
            pltpu.VMEM((2 * L,), jnp.float32),
        ],
        compiler_params=pltpu.CompilerParams(needs_layout_passes=False),
    )


def _topk_body(m_ref, k_ref, nneg_ref, out_ref):
    v = m_ref[...]
    neg = v >= 0.0
    k_eff = jnp.minimum(k_ref[0, 0], nneg_ref[...])

    def bit_step(i, t_bits):
        cand = jnp.bitwise_or(t_bits, lax.shift_left(jnp.int32(1), 30 - i))
        t = lax.bitcast_convert_type(cand, jnp.float32)
        cnt = jnp.sum(jnp.where(neg & (v >= t), 1.0, 0.0), axis=1, keepdims=True)
        return jnp.where(cnt >= k_eff, cand, t_bits)

    t_bits = lax.fori_loop(0, 31, bit_step, jnp.zeros((B, 1), jnp.int32))
    t = lax.bitcast_convert_type(t_bits, jnp.float32)
    sel = neg & (v > t)
    cgt = jnp.sum(jnp.where(sel, 1.0, 0.0), axis=1, keepdims=True)
    ssum = jnp.sum(jnp.where(sel, v, 0.0), axis=1, keepdims=True)
    rows = ssum + (k_eff - cgt) * t
    rows = jnp.where(k_eff > 0.0, rows, 0.0)
    out_ref[...] = jnp.sum(rows).reshape(1, 1)


def _topk_sum(m2d, kf, nneg):
    return pl.pallas_call(
        _topk_body,
        out_shape=jax.ShapeDtypeStruct((1, 1), jnp.float32),
    )(m2d, kf.reshape(1, 1), nneg.reshape(B, 1))[0, 0]


def kernel(confidence, locations, target_confidence, target_locations):
    ct = jnp.transpose(confidence, (2, 0, 1))
    lt = jnp.transpose(locations, (0, 2, 1))
    tlt = jnp.transpose(target_locations, (0, 2, 1))
    m1d, wpart, lpart = _prep(ct, lt, tlt, target_confidence)
    partials = _sc_main()(m1d)
    part = partials.reshape(B, 2, L)
    npos_rows = jnp.sum(part[:, 0, :], axis=1) - float(ROWP - P)
    num_pos = jnp.sum(npos_rows)
    negsum_rows = jnp.sum(part[:, 1, :], axis=1)
    wsum = jnp.sum(wpart)
    locsum = jnp.sum(lpart)
    kf = 3.0 * num_pos
    neg_contrib = lax.cond(
        kf >= float(P),
        lambda: jnp.sum(negsum_rows),
        lambda: _topk_sum(m1d.reshape(B, ROWP), kf, float(P) - npos_rows),
    )
    return (locsum + wsum + 0.05 * neg_contrib) / num_pos


# speedup vs baseline: 33.2643x; 1.0088x over previous
import functools

import jax
import jax.numpy as jnp
from jax import lax
from jax.experimental import pallas as pl
from jax.experimental.pallas import tpu as pltpu
from jax.experimental.pallas import tpu_sc as plsc

B = 32
P = 20000
C = 5
ROWP = 20224
NC = 2
NS = 16
L = 16
CHS = 4000
NCH = P // CHS
GRP = CHS // L
RPB = 8


def _prep_body(ct_ref, lt_ref, tlt_ref, lab_ref, m_ref, wpart_ref, lpart_ref):
    lab = lab_ref[...]
    pos = lab > 0
    posf = jnp.where(pos, 1.0, 0.0)
    x = [ct_ref[c] for c in range(C)]
    mx = jnp.maximum(jnp.maximum(jnp.maximum(x[0], x[1]),
                                 jnp.maximum(x[2], x[3])), x[4])
    s = (jnp.exp(x[0] - mx) + jnp.exp(x[1] - mx) + jnp.exp(x[2] - mx)
         + jnp.exp(x[3] - mx) + jnp.exp(x[4] - mx))
    lse = mx + jnp.log(s)
    xl = jnp.where(lab == 1, x[1], x[0])
    xl = jnp.where(lab == 2, x[2], xl)
    xl = jnp.where(lab == 3, x[3], xl)
    xl = jnp.where(lab == 4, x[4], xl)
    w = jnp.where(lab == 1, 1.0, 0.05)
    w = jnp.where(lab == 2, 5.0, w)
    w = jnp.where(lab == 3, 20.0, w)
    w = jnp.where(lab == 4, 10.0, w)
    wtot = jnp.sum(posf * (w * (lse - xl)))
    mvals = jnp.where(pos, -1.0, lse - x[0])
    mp = jnp.concatenate(
        [mvals, jnp.full((RPB, ROWP - P), -1.0, jnp.float32)],
        axis=1).astype(jnp.bfloat16)
    for r in range(RPB):
        m_ref[pl.ds(r * ROWP, ROWP)] = mp[r, :]
    d = lt_ref[...] - tlt_ref[...]
    sq = jnp.sum(d * d, axis=1)
    e = jnp.abs(jnp.exp(lt_ref[:, 2:4, :]) - jnp.exp(tlt_ref[:, 2:4, :]))
    ltot = jnp.sum((sq + jnp.sum(e, axis=1)) * posf)
    lane0 = jax.lax.broadcasted_iota(jnp.int32, (1, 1, 128), 2) == 0
    wpart_ref[...] = jnp.where(lane0, wtot, 0.0)
    lpart_ref[...] = jnp.where(lane0, ltot, 0.0)


def _prep(ct, lt, tlt, lab):
    nsteps = B // RPB
    return pl.pallas_call(
        _prep_body,
        grid=(nsteps,),
        in_specs=[
            pl.BlockSpec((C, RPB, P), lambda i: (0, i, 0)),
            pl.BlockSpec((RPB, 4, P), lambda i: (i, 0, 0)),
            pl.BlockSpec((RPB, 4, P), lambda i: (i, 0, 0)),
            pl.BlockSpec((RPB, P), lambda i: (i, 0)),
        ],
        out_specs=[
            pl.BlockSpec((RPB * ROWP,), lambda i: (i,)),
            pl.BlockSpec((1, 1, 128), lambda i: (i, 0, 0)),
            pl.BlockSpec((1, 1, 128), lambda i: (i, 0, 0)),
        ],
        out_shape=[
            jax.ShapeDtypeStruct((B * ROWP,), jnp.bfloat16),
            jax.ShapeDtypeStruct((nsteps, 1, 128), jnp.float32),
            jax.ShapeDtypeStruct((nsteps, 1, 128), jnp.float32),
        ],
    )(ct, lt, tlt, lab)


_UNROLL = 2
_NITER = ROWP // (2 * L * _UNROLL)


def _sc_body(m_hbm, part_hbm, mbuf, obuf):
    wid = lax.axis_index("s") * NC + lax.axis_index("c")
    zero = jnp.zeros((L,), jnp.float32)
    pltpu.sync_copy(m_hbm.at[pl.ds(wid * ROWP, ROWP)], mbuf)

    def group(g, carry):
        base = g * (2 * L * _UNROLL)
        out = []
        for u in range(_UNROLL):
            npos_u, negsum_u = carry[2 * u], carry[2 * u + 1]
            vb = mbuf[pl.ds(base + u * (2 * L), 2 * L)]
            va, vc = plsc.unpack(vb, format=plsc.PackFormat.INTERLEAVED)
            for v in (va, vc):
                isneg = v >= 0.0
                npos_u = npos_u + jnp.where(isneg, 0.0, 1.0)
                negsum_u = negsum_u + jnp.where(isneg, v, 0.0)
            out.append(npos_u)
            out.append(negsum_u)
        return tuple(out)

    acc = lax.fori_loop(0, _NITER, group, (zero,) * (2 * _UNROLL))
    npos_a = acc[0] + acc[2]
    negsum_a = acc[1] + acc[3]
    obuf[pl.ds(0, L)] = npos_a
    obuf[pl.ds(L, L)] = negsum_a
    pltpu.sync_copy(obuf, part_hbm.at[pl.ds(wid * (2 * L), 2 * L)])


@functools.cache
def _sc_main():
    return pl.kernel(
        _sc_body,
        out_type=jax.ShapeDtypeStruct((B * 2 * L,), jnp.float32),
        mesh=plsc.VectorSubcoreMesh(core_axis_name="c", subcore_axis_name="s",
                                    num_cores=NC, num_subcores=NS),
        scratch_types=[
            pltpu.VMEM((ROWP,), jnp.bfloat16),
            pltpu.VMEM((2 * L,), jnp.float32),
        ],
        compiler_params=pltpu.CompilerParams(needs_layout_passes=False),
    )


def _topk_body(m_ref, k_ref, nneg_ref, out_ref):
    v = m_ref[...].astype(jnp.float32)
    neg = v >= 0.0
    k_eff = jnp.minimum(k_ref[0, 0], nneg_ref[...])

    def bit_step(i, t_bits):
        cand = jnp.bitwise_or(t_bits, lax.shift_left(jnp.int32(1), 30 - i))
        t = lax.bitcast_convert_type(cand, jnp.float32)
        cnt = jnp.sum(jnp.where(neg & (v >= t), 1.0, 0.0), axis=1, keepdims=True)
        return jnp.where(cnt >= k_eff, cand, t_bits)

    t_bits = lax.fori_loop(0, 31, bit_step, jnp.zeros((B, 1), jnp.int32))
    t = lax.bitcast_convert_type(t_bits, jnp.float32)
    sel = neg & (v > t)
    cgt = jnp.sum(jnp.where(sel, 1.0, 0.0), axis=1, keepdims=True)
    ssum = jnp.sum(jnp.where(sel, v, 0.0), axis=1, keepdims=True)
    rows = ssum + (k_eff - cgt) * t
    rows = jnp.where(k_eff > 0.0, rows, 0.0)
    out_ref[...] = jnp.sum(rows).reshape(1, 1)


def _topk_sum(m2d, kf, nneg):
    return pl.pallas_call(
        _topk_body,
        out_shape=jax.ShapeDtypeStruct((1, 1), jnp.float32),
    )(m2d, kf.reshape(1, 1), nneg.reshape(B, 1))[0, 0]


def kernel(confidence, locations, target_confidence, target_locations):
    ct = jnp.transpose(confidence, (2, 0, 1))
    lt = jnp.transpose(locations, (0, 2, 1))
    tlt = jnp.transpose(target_locations, (0, 2, 1))
    m1d, wpart, lpart = _prep(ct, lt, tlt, target_confidence)
    partials = _sc_main()(m1d)
    part = partials.reshape(B, 2, L)
    npos_rows = jnp.sum(part[:, 0, :], axis=1) - float(ROWP - P)
    num_pos = jnp.sum(npos_rows)
    negsum_rows = jnp.sum(part[:, 1, :], axis=1)
    wsum = jnp.sum(wpart)
    locsum = jnp.sum(lpart)
    kf = 3.0 * num_pos
    neg_contrib = lax.cond(
        kf >= float(P),
        lambda: jnp.sum(negsum_rows),
        lambda: _topk_sum(m1d.reshape(B, ROWP), kf, float(P) - npos_rows),
    )
    return (locsum + wsum + 0.05 * neg_contrib) / num_pos
